# baseline (speedup 1.0000x reference)
import functools

import jax
import jax.numpy as jnp
from jax import lax
from jax.experimental import pallas as pl
from jax.experimental.pallas import tpu as pltpu
from jax.experimental.pallas import tpu_sc as plsc

B, S, H, A, N = 1024, 128, 64, 18, 100000
P_NEIGHBORS = 50
DELTA = 1e-3

GROUP = 128
N_PAD = 100352
N_GROUPS = N_PAD // GROUP
CHUNK = 2048
G_PER_CHUNK = CHUNK // GROUP
N_STEPS = N_PAD // CHUNK
PAD_KEY = 1.0e6


def _encoder_d2_kernel(x_ref, w1_ref, b1_ref, wp_ref, bp_ref, keys_ref,
                       h_ref, logits_ref, d2_ref, g_ref):
    j = pl.program_id(0)

    @pl.when(j == 0)
    def _():
        h0 = jnp.maximum(
            jnp.dot(x_ref[...], w1_ref[...].T, preferred_element_type=jnp.float32)
            + b1_ref[...], 0.0)
        h_ref[...] = h0
        logits_ref[...] = (
            jnp.dot(h0, wp_ref[...].T, preferred_element_type=jnp.float32)
            + bp_ref[...])

    h = h_ref[...]
    hh = jnp.sum(h * h, axis=1, keepdims=True)
    keys = keys_ref[...]
    kk = jnp.sum(keys * keys, axis=1)[None, :]
    s = jnp.dot(h, keys.T, preferred_element_type=jnp.float32)
    d2 = hh - 2.0 * s + kk
    d2_ref[...] = d2
    nb = d2.shape[0]
    gmin = jnp.min(d2.reshape(nb, G_PER_CHUNK, GROUP), axis=-1)
    g_ref[...] = gmin[None]


def _encoder_d2(x, W1, b1, Wp, bp, keys_pad, nb):
    return pl.pallas_call(
        _encoder_d2_kernel,
        grid=(N_STEPS,),
        in_specs=[
            pl.BlockSpec((nb, S), lambda j: (0, 0)),
            pl.BlockSpec((H, S), lambda j: (0, 0)),
            pl.BlockSpec((1, H), lambda j: (0, 0)),
            pl.BlockSpec((A, H), lambda j: (0, 0)),
            pl.BlockSpec((1, A), lambda j: (0, 0)),
            pl.BlockSpec((CHUNK, H), lambda j: (j, 0)),
        ],
        out_specs=[
            pl.BlockSpec((nb, H), lambda j: (0, 0)),
            pl.BlockSpec((nb, A), lambda j: (0, 0)),
            pl.BlockSpec((nb, CHUNK), lambda j: (0, j)),
            pl.BlockSpec((1, nb, G_PER_CHUNK), lambda j: (j, 0, 0)),
        ],
        out_shape=[
            jax.ShapeDtypeStruct((nb, H), jnp.float32),
            jax.ShapeDtypeStruct((nb, A), jnp.float32),
            jax.ShapeDtypeStruct((nb, N_PAD), jnp.float32),
            jax.ShapeDtypeStruct((N_STEPS, nb, G_PER_CHUNK), jnp.float32),
        ],
    )(x, W1, b1, Wp, bp, keys_pad)


NC, NS, L = 2, 16, 16
NW = NC * NS
RPW = B // NW
P = P_NEIGHBORS
ECAP = 1024
GCHUNK = 64
EPHYS = ECAP + GROUP + L
NV_G = N_GROUPS // L
INF = float("inf")
PIVOT_ITERS = 7


def _iota():
    return lax.iota(jnp.int32, L)


def _pcount(mask):
    return plsc.all_reduce_population_count(mask)[0]


def _red_vec(x, op, tmp):
    for r in (8, 4, 2, 1):
        tmp[...] = x
        x = op(x, plsc.load_gather(tmp, [_iota() ^ r]))
    return x


def _red(x, op, tmp):
    return _red_vec(x, op, tmp)[0]


def _recip(x):
    u = plsc.bitcast(x, jnp.uint32)
    r = plsc.bitcast(jnp.uint32(0x7EF311C3) - u, jnp.float32)
    for _ in range(3):
        r = r * (2.0 - x * r)
    return r


def _prefix_sum_incl(v, tmp):
    cum = v
    for r in (1, 2, 4, 8):
        tmp[...] = cum
        sh = plsc.load_gather(tmp, [jnp.maximum(_iota() - r, 0)])
        cum = cum + jnp.where(_iota() >= r, sh, 0)
    return cum


def _copy_buf(src, dst, m):
    def body(v, _):
        dst[pl.ds(v * L, L)] = src[pl.ds(v * L, L)]
        return 0
    lax.fori_loop(0, (m + L - 1) // L, body, 0)


def _kth_smallest(wa, wb, m0, k0, tmpf):
    def pivot_round(_, carry):
        m, k, t, done = carry

        def p1(v, acc):
            sv, mnv, mxv = acc
            x = wa[pl.ds(v * L, L)]
            valid = (v * L + _iota()) < m
            sv = sv + jnp.where(valid, x, 0.0)
            mnv = jnp.minimum(mnv, jnp.where(valid, x, INF))
            mxv = jnp.maximum(mxv, jnp.where(valid, x, -INF))
            return (sv, mnv, mxv)
        sv, mnv, mxv = lax.fori_loop(
            0, (m + L - 1) // L, p1,
            (jnp.zeros((L,), jnp.float32), jnp.full((L,), INF, jnp.float32),
             jnp.full((L,), -INF, jnp.float32)))
        msplat = jnp.full((L,), m.astype(jnp.float32))
        mean = (_red_vec(sv, jnp.add, tmpf) * _recip(msplat))[0]
        mn = _red(mnv, jnp.minimum, tmpf)
        mx = _red(mxv, jnp.maximum, tmpf)

        def p2(v, acc):
            clt, ceq = acc
            x = wa[pl.ds(v * L, L)]
            valid = (v * L + _iota()) < m
            clt = clt + _pcount(valid & (x < mean))
            ceq = ceq + _pcount(valid & (x == mn))
            return (clt, ceq)
        cnt_lt, cnt_eq = lax.fori_loop(
            0, (m + L - 1) // L, p2, (jnp.int32(0), jnp.int32(0)))

        fallback = (cnt_lt == 0) | (cnt_lt == m)
        now_done = (m == k) | (fallback & (k <= cnt_eq))
        t = jnp.where(done, t, jnp.where(m == k, mx, mn))
        done2 = done | now_done

        use_gt = fallback
        use_lt = jnp.logical_not(fallback) & (cnt_lt >= k)

        def p3(v, cnt):
            x = wa[pl.ds(v * L, L)]
            valid = (v * L + _iota()) < m
            keep = jnp.where(use_gt, x > mn,
                             jnp.where(use_lt, x < mean, x >= mean))
            mask = valid & keep & jnp.logical_not(done2)
            plsc.store_compressed(wb.at[pl.ds(cnt, L)], x, mask=mask)
            return cnt + _pcount(mask)
        nm = lax.fori_loop(0, (m + L - 1) // L, p3, jnp.int32(0))
        _copy_buf(wb, wa, nm)

        k2 = jnp.where(done2 | use_lt, k,
                       jnp.where(use_gt, k - cnt_eq, k - cnt_lt))
        m2 = jnp.where(done2, m, nm)
        return (m2, k2, t, done2)

    m0 = jnp.int32(m0)
    k0 = jnp.int32(k0)
    m, k, t, done = lax.fori_loop(
        0, PIVOT_ITERS, pivot_round,
        (m0, k0, jnp.float32(0.0), jnp.bool_(False)))

    sign = jnp.uint32(0x80000000)

    def mapb(v, _):
        x = wa[pl.ds(v * L, L)]
        u = plsc.bitcast(x, jnp.uint32)
        mk = jnp.where(u >= sign, ~u, u | sign)
        wb[pl.ds(v * L, L)] = plsc.bitcast(mk, jnp.float32)
        return 0
    lax.fori_loop(0, (m + L - 1) // L, mapb, 0)

    def bit_round(i, kk_acc):
        cand = kk_acc | (jnp.uint32(1) << (jnp.uint32(31) - i.astype(jnp.uint32)))

        def pc(v, acc):
            mk = plsc.bitcast(wb[pl.ds(v * L, L)], jnp.uint32)
            valid = (v * L + _iota()) < m
            return acc + _pcount(valid & (mk < cand))
        cnt = lax.fori_loop(0, (m + L - 1) // L, pc, jnp.int32(0))
        return jnp.where(cnt < k, cand, kk_acc)

    kbits = lax.fori_loop(0, 32, bit_round, jnp.uint32(0))
    kv = jnp.full((L,), kbits)
    uv = jnp.where(kv >= sign, kv & jnp.uint32(0x7FFFFFFF), ~kv)
    t_bin = plsc.bitcast(uv, jnp.float32)[0]
    return jnp.where(done, t, t_bin)


def _select_exact(ev, ei, en, t2, need, fv, fi, tmpi):
    def pc(v, acc):
        x = ev[pl.ds(v * L, L)]
        valid = (v * L + _iota()) < en
        return acc + _pcount(valid & (x < t2))
    cnt_lt = lax.fori_loop(0, (en + L - 1) // L, pc, jnp.int32(0))
    need_eq = need - cnt_lt

    def body(v, carry):
        fc, eqs = carry
        x = ev[pl.ds(v * L, L)]
        ix = ei[pl.ds(v * L, L)]
        valid = (v * L + _iota()) < en
        m_lt = valid & (x < t2)
        m_eq = valid & (x == t2)
        eq_rank = _prefix_sum_incl(m_eq.astype(jnp.int32), tmpi)
        take_eq = m_eq & ((eqs + eq_rank) <= need_eq)
        mask = m_lt | take_eq
        plsc.store_compressed(fv.at[pl.ds(fc, L)], x, mask=mask)
        plsc.store_compressed(fi.at[pl.ds(fc, L)], ix, mask=mask)
        fc = fc + _pcount(mask)
        eqs = eqs + _pcount(m_eq)
        return (fc, eqs)

    lax.fori_loop(0, (en + L - 1) // L, body, (jnp.int32(0), jnp.int32(0)))


def _make_sc_select(nb):
    mesh = plsc.VectorSubcoreMesh(core_axis_name="c", subcore_axis_name="s",
                                  num_cores=NC, num_subcores=NS)

    @functools.partial(
        pl.kernel,
        out_type=jax.ShapeDtypeStruct((nb,), jnp.float32),
        mesh=mesh,
        compiler_params=pltpu.CompilerParams(needs_layout_passes=False),
        scratch_types=[
            pltpu.VMEM((N_STEPS, 1, G_PER_CHUNK), jnp.float32),
            pltpu.VMEM((N_GROUPS + L,), jnp.float32),
            pltpu.VMEM((EPHYS,), jnp.float32),
            pltpu.VMEM((EPHYS,), jnp.float32),
            pltpu.VMEM((N_GROUPS + L,), jnp.int32),
            pltpu.VMEM((N_GROUPS + L,), jnp.int32),
            pltpu.VMEM((GCHUNK,), jnp.int32),
            pltpu.VMEM((GCHUNK, GROUP), jnp.float32),
            pltpu.VMEM((EPHYS,), jnp.float32),
            pltpu.VMEM((EPHYS,), jnp.int32),
            pltpu.VMEM((5 * L,), jnp.float32),
            pltpu.VMEM((5 * L,), jnp.int32),
            pltpu.VMEM((N_PAD,), jnp.float32),
            pltpu.VMEM((3 * L,), jnp.float32),
            pltpu.VMEM((GCHUNK + L,), jnp.int32),
            pltpu.VMEM((L,), jnp.float32),
            pltpu.VMEM((L,), jnp.int32),
            pltpu.SemaphoreType.DMA,
        ],
    )
    def sc_select(g3, d2r, vals1d, out, grow, wa, wb, wa2, sel_rid, sel_gid,
                  idx_rows, cand_d2, elem_v, elem_i, fv, fi, valsv, out_v,
                  gidbuf, tmpf, tmpi, sem):
        rpw = nb // NW
        wid = lax.axis_index("s") * NC + lax.axis_index("c")
        b_base = wid * rpw
        pltpu.sync_copy(vals1d, valsv)

        def row_body(i, _):
            b = b_base + i
            pltpu.sync_copy(g3.at[:, pl.ds(b, 1), :], grow)

            def init_wa(j, _):
                wa[pl.ds(j * L, L)] = grow[j, 0]
                return 0
            lax.fori_loop(0, NV_G, init_wa, 0)

            t = _kth_smallest(wa, wb, N_GROUPS, P, tmpf)

            def sel_body(j, ns):
                x = grow[j, 0]
                rid = b * N_GROUPS + j * L + _iota()
                gid = j * L + _iota()
                mask = x <= t
                plsc.store_compressed(sel_rid.at[pl.ds(ns, L)], rid, mask=mask)
                plsc.store_compressed(sel_gid.at[pl.ds(ns, L)], gid, mask=mask)
                return ns + _pcount(mask)
            ns = lax.fori_loop(0, NV_G, sel_body, jnp.int32(0))

            rid_fill = sel_rid[pl.ds(0, L)][0]

            def chunk_body(c, en):
                for q in range(GCHUNK // L):
                    lanebase = c * GCHUNK + q * L
                    rid = sel_rid[pl.ds(lanebase, L)]
                    gid = sel_gid[pl.ds(lanebase, L)]
                    valid = (lanebase + _iota()) < ns
                    idx_rows[pl.ds(q * L, L)] = jnp.where(valid, rid, rid_fill)
                    gidbuf[pl.ds(q * L, L)] = gid
                pltpu.async_copy(d2r.at[idx_rows], cand_d2, sem).wait()

                n_take = jnp.minimum(ns - c * GCHUNK, GCHUNK)

                def group_body(gi, en2):
                    gid = gidbuf[pl.ds(gi, L)][0]
                    ebase = gid * GROUP
                    for sub in range(GROUP // L):
                        x = cand_d2[gi, pl.ds(sub * L, L)]
                        eidx = ebase + sub * L + _iota()
                        mask = x <= t
                        plsc.store_compressed(elem_v.at[pl.ds(en2, L)], x,
                                              mask=mask)
                        plsc.store_compressed(elem_i.at[pl.ds(en2, L)], eidx,
                                              mask=mask)
                        en2 = en2 + _pcount(mask)
                    return jnp.minimum(en2, jnp.int32(ECAP))

                return lax.fori_loop(0, n_take, group_body, en)

            en = lax.fori_loop(0, (ns + GCHUNK - 1) // GCHUNK,
                               chunk_body, jnp.int32(0))

            for q in range(5):
                fv[pl.ds(q * L, L)] = jnp.full((L,), INF, jnp.float32)
                fi[pl.ds(q * L, L)] = jnp.zeros((L,), jnp.int32)
            _copy_buf(elem_v, wa2, en)
            t2 = _kth_smallest(wa2, wb, en, P, tmpf)
            _select_exact(elem_v, elem_i, en, t2, P, fv, fi, tmpi)

            acc = jnp.zeros((L,), jnp.float32)
            wsum = jnp.zeros((L,), jnp.float32)
            for q in range(4):
                d = fv[pl.ds(q * L, L)]
                lane_ok = (q * L + _iota()) < P
                w = jnp.where(lane_ok, _recip(d + DELTA), 0.0)
                v = plsc.load_gather(valsv, [fi[pl.ds(q * L, L)]])
                acc = acc + w * v
                wsum = wsum + w
            accs = _red_vec(acc, jnp.add, tmpf)
            wss = _red_vec(wsum, jnp.add, tmpf)
            value_v = accs * _recip(wss)
            plsc.store_compressed(out_v.at[pl.ds(i, L)], value_v,
                                  mask=_iota() == 0)
            return 0

        lax.fori_loop(0, rpw, row_body, 0)
        pltpu.sync_copy(out_v.at[pl.ds(0, rpw)], out.at[pl.ds(b_base, rpw)])

    return sc_select


NSPLIT = 4
NB = B // NSPLIT
_sc_select = _make_sc_select(NB)


def kernel(x, W1, b1, Wp, bp, dnd_keys, dnd_vals):
    keys_pad = jnp.pad(dnd_keys, ((0, N_PAD - N), (0, 0)),
                       constant_values=PAD_KEY)
    vals1d = jnp.pad(dnd_vals[:, 0], (0, N_PAD - N))
    hs, ls, vs = [], [], []
    for p in range(NSPLIT):
        h, logits, d2, g3 = _encoder_d2(
            x[p * NB:(p + 1) * NB], W1, b1.reshape(1, H), Wp,
            bp.reshape(1, A), keys_pad, NB)
        d2r = d2.reshape(NB * N_GROUPS, GROUP)
        value = _sc_select(g3, d2r, vals1d)
        hs.append(h)
        ls.append(logits)
        vs.append(value)
    h = jnp.concatenate(hs, axis=0)
    logits = jnp.concatenate(ls, axis=0)
    value = jnp.concatenate(vs, axis=0)
    return (logits, value.reshape(B, 1), h)


# speedup vs baseline: 1.0328x; 1.0328x over previous
import functools

import jax
import jax.numpy as jnp
from jax import lax
from jax.experimental import pallas as pl
from jax.experimental.pallas import tpu as pltpu
from jax.experimental.pallas import tpu_sc as plsc

B, S, H, A, N = 1024, 128, 64, 18, 100000
P_NEIGHBORS = 50
DELTA = 1e-3

GROUP = 128
N_PAD = 100352
N_GROUPS = N_PAD // GROUP
CHUNK = 2048
G_PER_CHUNK = CHUNK // GROUP
N_STEPS = N_PAD // CHUNK
PAD_KEY = 1.0e6


def _encoder_d2_kernel(x_ref, w1_ref, b1_ref, wp_ref, bp_ref, keys_ref,
                       h_ref, logits_ref, d2_ref, g_ref):
    j = pl.program_id(0)

    @pl.when(j == 0)
    def _():
        h0 = jnp.maximum(
            jnp.dot(x_ref[...], w1_ref[...].T, preferred_element_type=jnp.float32)
            + b1_ref[...], 0.0)
        h_ref[...] = h0
        logits_ref[...] = (
            jnp.dot(h0, wp_ref[...].T, preferred_element_type=jnp.float32)
            + bp_ref[...])

    h = h_ref[...]
    hh = jnp.sum(h * h, axis=1, keepdims=True)
    keys = keys_ref[...]
    kk = jnp.sum(keys * keys, axis=1)[None, :]
    s = jnp.dot(h, keys.T, preferred_element_type=jnp.float32)
    d2 = hh - 2.0 * s + kk
    d2_ref[...] = d2
    nb = d2.shape[0]
    gmin = jnp.min(d2.reshape(nb, G_PER_CHUNK, GROUP), axis=-1)
    g_ref[...] = gmin[None]


def _encoder_d2(x, W1, b1, Wp, bp, keys_pad, nb):
    return pl.pallas_call(
        _encoder_d2_kernel,
        grid=(N_STEPS,),
        in_specs=[
            pl.BlockSpec((nb, S), lambda j: (0, 0)),
            pl.BlockSpec((H, S), lambda j: (0, 0)),
            pl.BlockSpec((1, H), lambda j: (0, 0)),
            pl.BlockSpec((A, H), lambda j: (0, 0)),
            pl.BlockSpec((1, A), lambda j: (0, 0)),
            pl.BlockSpec((CHUNK, H), lambda j: (j, 0)),
        ],
        out_specs=[
            pl.BlockSpec((nb, H), lambda j: (0, 0)),
            pl.BlockSpec((nb, A), lambda j: (0, 0)),
            pl.BlockSpec((nb, CHUNK), lambda j: (0, j)),
            pl.BlockSpec((1, nb, G_PER_CHUNK), lambda j: (j, 0, 0)),
        ],
        out_shape=[
            jax.ShapeDtypeStruct((nb, H), jnp.float32),
            jax.ShapeDtypeStruct((nb, A), jnp.float32),
            jax.ShapeDtypeStruct((nb, N_PAD), jnp.float32),
            jax.ShapeDtypeStruct((N_STEPS, nb, G_PER_CHUNK), jnp.float32),
        ],
    )(x, W1, b1, Wp, bp, keys_pad)


NC, NS, L = 2, 16, 16
NW = NC * NS
RPW = B // NW
P = P_NEIGHBORS
ECAP = 1024
GCHUNK = 64
EPHYS = ECAP + GROUP + L
NV_G = N_GROUPS // L
N_GROUPS_REAL = (N + GROUP - 1) // GROUP
INF = float("inf")
PIVOT_ITERS = 7


def _iota():
    return lax.iota(jnp.int32, L)


def _pcount(mask):
    return plsc.all_reduce_population_count(mask)[0]


def _red_vec(x, op, tmp):
    for r in (8, 4, 2, 1):
        tmp[...] = x
        x = op(x, plsc.load_gather(tmp, [_iota() ^ r]))
    return x


def _red(x, op, tmp):
    return _red_vec(x, op, tmp)[0]


def _recip(x):
    u = plsc.bitcast(x, jnp.uint32)
    r = plsc.bitcast(jnp.uint32(0x7EF311C3) - u, jnp.float32)
    for _ in range(3):
        r = r * (2.0 - x * r)
    return r


def _prefix_sum_incl(v, tmp):
    cum = v
    for r in (1, 2, 4, 8):
        tmp[...] = cum
        sh = plsc.load_gather(tmp, [jnp.maximum(_iota() - r, 0)])
        cum = cum + jnp.where(_iota() >= r, sh, 0)
    return cum


def _copy_buf(src, dst, m):
    def body(v, _):
        dst[pl.ds(v * L, L)] = src[pl.ds(v * L, L)]
        return 0
    lax.fori_loop(0, (m + L - 1) // L, body, 0)


def _kth_smallest(wa, wb, m0, k0, tmpf):
    def pivot_round(_, carry):
        m, k, t, done = carry

        def p1(v, acc):
            sv, mnv, mxv = acc
            x = wa[pl.ds(v * L, L)]
            valid = (v * L + _iota()) < m
            sv = sv + jnp.where(valid, x, 0.0)
            mnv = jnp.minimum(mnv, jnp.where(valid, x, INF))
            mxv = jnp.maximum(mxv, jnp.where(valid, x, -INF))
            return (sv, mnv, mxv)
        sv, mnv, mxv = lax.fori_loop(
            0, (m + L - 1) // L, p1,
            (jnp.zeros((L,), jnp.float32), jnp.full((L,), INF, jnp.float32),
             jnp.full((L,), -INF, jnp.float32)))
        msplat = jnp.full((L,), m.astype(jnp.float32))
        mean = (_red_vec(sv, jnp.add, tmpf) * _recip(msplat))[0]
        mn = _red(mnv, jnp.minimum, tmpf)
        mx = _red(mxv, jnp.maximum, tmpf)

        def p2(v, acc):
            clt, ceq = acc
            x = wa[pl.ds(v * L, L)]
            valid = (v * L + _iota()) < m
            clt = clt + _pcount(valid & (x < mean))
            ceq = ceq + _pcount(valid & (x == mn))
            return (clt, ceq)
        cnt_lt, cnt_eq = lax.fori_loop(
            0, (m + L - 1) // L, p2, (jnp.int32(0), jnp.int32(0)))

        fallback = (cnt_lt == 0) | (cnt_lt == m)
        now_done = (m == k) | (fallback & (k <= cnt_eq))
        t = jnp.where(done, t, jnp.where(m == k, mx, mn))
        done2 = done | now_done

        use_gt = fallback
        use_lt = jnp.logical_not(fallback) & (cnt_lt >= k)

        def p3(v, cnt):
            x = wa[pl.ds(v * L, L)]
            valid = (v * L + _iota()) < m
            keep = jnp.where(use_gt, x > mn,
                             jnp.where(use_lt, x < mean, x >= mean))
            mask = valid & keep & jnp.logical_not(done2)
            plsc.store_compressed(wb.at[pl.ds(cnt, L)], x, mask=mask)
            return cnt + _pcount(mask)
        nm = lax.fori_loop(0, (m + L - 1) // L, p3, jnp.int32(0))
        _copy_buf(wb, wa, nm)

        k2 = jnp.where(done2 | use_lt, k,
                       jnp.where(use_gt, k - cnt_eq, k - cnt_lt))
        m2 = jnp.where(done2, m, nm)
        return (m2, k2, t, done2)

    m0 = jnp.int32(m0)
    k0 = jnp.int32(k0)
    m, k, t, done = lax.fori_loop(
        0, PIVOT_ITERS, pivot_round,
        (m0, k0, jnp.float32(0.0), jnp.bool_(False)))

    sign = jnp.uint32(0x80000000)

    def mapb(v, _):
        x = wa[pl.ds(v * L, L)]
        u = plsc.bitcast(x, jnp.uint32)
        mk = jnp.where(u >= sign, ~u, u | sign)
        wb[pl.ds(v * L, L)] = plsc.bitcast(mk, jnp.float32)
        return 0
    lax.fori_loop(0, (m + L - 1) // L, mapb, 0)

    def bit_round(i, kk_acc):
        cand = kk_acc | (jnp.uint32(1) << (jnp.uint32(31) - i.astype(jnp.uint32)))

        def pc(v, acc):
            mk = plsc.bitcast(wb[pl.ds(v * L, L)], jnp.uint32)
            valid = (v * L + _iota()) < m
            return acc + _pcount(valid & (mk < cand))
        cnt = lax.fori_loop(0, (m + L - 1) // L, pc, jnp.int32(0))
        return jnp.where(cnt < k, cand, kk_acc)

    kbits = lax.fori_loop(0, 32, bit_round, jnp.uint32(0))
    kv = jnp.full((L,), kbits)
    uv = jnp.where(kv >= sign, kv & jnp.uint32(0x7FFFFFFF), ~kv)
    t_bin = plsc.bitcast(uv, jnp.float32)[0]
    return jnp.where(done, t, t_bin)


def _select_exact(ev, ei, en, t2, need, fv, fi, tmpi):
    def pc(v, acc):
        x = ev[pl.ds(v * L, L)]
        valid = (v * L + _iota()) < en
        return acc + _pcount(valid & (x < t2))
    cnt_lt = lax.fori_loop(0, (en + L - 1) // L, pc, jnp.int32(0))
    need_eq = need - cnt_lt

    def body(v, carry):
        fc, eqs = carry
        x = ev[pl.ds(v * L, L)]
        ix = ei[pl.ds(v * L, L)]
        valid = (v * L + _iota()) < en
        m_lt = valid & (x < t2)
        m_eq = valid & (x == t2)
        eq_rank = _prefix_sum_incl(m_eq.astype(jnp.int32), tmpi)
        take_eq = m_eq & ((eqs + eq_rank) <= need_eq)
        mask = m_lt | take_eq
        plsc.store_compressed(fv.at[pl.ds(fc, L)], x, mask=mask)
        plsc.store_compressed(fi.at[pl.ds(fc, L)], ix, mask=mask)
        fc = fc + _pcount(mask)
        eqs = eqs + _pcount(m_eq)
        return (fc, eqs)

    lax.fori_loop(0, (en + L - 1) // L, body, (jnp.int32(0), jnp.int32(0)))


def _make_sc_select(nb):
    mesh = plsc.VectorSubcoreMesh(core_axis_name="c", subcore_axis_name="s",
                                  num_cores=NC, num_subcores=NS)

    @functools.partial(
        pl.kernel,
        out_type=jax.ShapeDtypeStruct((nb,), jnp.float32),
        mesh=mesh,
        compiler_params=pltpu.CompilerParams(needs_layout_passes=False),
        scratch_types=[
            pltpu.VMEM((N_STEPS, 1, G_PER_CHUNK), jnp.float32),
            pltpu.VMEM((N_GROUPS + L,), jnp.float32),
            pltpu.VMEM((EPHYS,), jnp.float32),
            pltpu.VMEM((EPHYS,), jnp.float32),
            pltpu.VMEM((N_GROUPS + L,), jnp.int32),
            pltpu.VMEM((N_GROUPS + L,), jnp.int32),
            pltpu.VMEM((GCHUNK,), jnp.int32),
            pltpu.VMEM((GCHUNK, GROUP), jnp.float32),
            pltpu.VMEM((EPHYS,), jnp.float32),
            pltpu.VMEM((EPHYS,), jnp.int32),
            pltpu.VMEM((5 * L,), jnp.float32),
            pltpu.VMEM((5 * L,), jnp.int32),
            pltpu.VMEM((N_PAD,), jnp.float32),
            pltpu.VMEM((3 * L,), jnp.float32),
            pltpu.VMEM((GCHUNK + L,), jnp.int32),
            pltpu.VMEM((L,), jnp.float32),
            pltpu.VMEM((L,), jnp.int32),
            pltpu.SemaphoreType.DMA,
        ],
    )
    def sc_select(g3, d2r, vals1d, out, grow, wa, wb, wa2, sel_rid, sel_gid,
                  idx_rows, cand_d2, elem_v, elem_i, fv, fi, valsv, out_v,
                  gidbuf, tmpf, tmpi, sem):
        rpw = nb // NW
        wid = lax.axis_index("s") * NC + lax.axis_index("c")
        b_base = wid * rpw
        pltpu.sync_copy(vals1d, valsv)

        def row_body(i, _):
            b = b_base + i
            pltpu.sync_copy(g3.at[:, pl.ds(b, 1), :], grow)

            def init_wa(j, _):
                wa[pl.ds(j * L, L)] = grow[j, 0]
                return 0
            lax.fori_loop(0, NV_G, init_wa, 0)

            t = _kth_smallest(wa, wb, N_GROUPS_REAL, P, tmpf)

            def sel_body(j, ns):
                x = grow[j, 0]
                rid = b * N_GROUPS + j * L + _iota()
                gid = j * L + _iota()
                mask = x <= t
                plsc.store_compressed(sel_rid.at[pl.ds(ns, L)], rid, mask=mask)
                plsc.store_compressed(sel_gid.at[pl.ds(ns, L)], gid, mask=mask)
                return ns + _pcount(mask)
            ns = lax.fori_loop(0, NV_G, sel_body, jnp.int32(0))

            rid_fill = sel_rid[pl.ds(0, L)][0]

            def chunk_body(c, en):
                for q in range(GCHUNK // L):
                    lanebase = c * GCHUNK + q * L
                    rid = sel_rid[pl.ds(lanebase, L)]
                    gid = sel_gid[pl.ds(lanebase, L)]
                    valid = (lanebase + _iota()) < ns
                    idx_rows[pl.ds(q * L, L)] = jnp.where(valid, rid, rid_fill)
                    gidbuf[pl.ds(q * L, L)] = gid
                pltpu.async_copy(d2r.at[idx_rows], cand_d2, sem).wait()

                n_take = jnp.minimum(ns - c * GCHUNK, GCHUNK)

                def group_body(gi, en2):
                    gid = gidbuf[pl.ds(gi, L)][0]
                    ebase = gid * GROUP
                    for sub in range(GROUP // L):
                        x = cand_d2[gi, pl.ds(sub * L, L)]
                        eidx = ebase + sub * L + _iota()
                        mask = x <= t
                        plsc.store_compressed(elem_v.at[pl.ds(en2, L)], x,
                                              mask=mask)
                        plsc.store_compressed(elem_i.at[pl.ds(en2, L)], eidx,
                                              mask=mask)
                        en2 = en2 + _pcount(mask)
                    return jnp.minimum(en2, jnp.int32(ECAP))

                return lax.fori_loop(0, n_take, group_body, en)

            en = lax.fori_loop(0, (ns + GCHUNK - 1) // GCHUNK,
                               chunk_body, jnp.int32(0))

            for q in range(5):
                fv[pl.ds(q * L, L)] = jnp.full((L,), INF, jnp.float32)
                fi[pl.ds(q * L, L)] = jnp.zeros((L,), jnp.int32)
            _copy_buf(elem_v, wa2, en)
            t2 = _kth_smallest(wa2, wb, en, P, tmpf)
            _select_exact(elem_v, elem_i, en, t2, P, fv, fi, tmpi)

            acc = jnp.zeros((L,), jnp.float32)
            wsum = jnp.zeros((L,), jnp.float32)
            for q in range(4):
                d = fv[pl.ds(q * L, L)]
                lane_ok = (q * L + _iota()) < P
                w = jnp.where(lane_ok, _recip(d + DELTA), 0.0)
                v = plsc.load_gather(valsv, [fi[pl.ds(q * L, L)]])
                acc = acc + w * v
                wsum = wsum + w
            accs = _red_vec(acc, jnp.add, tmpf)
            wss = _red_vec(wsum, jnp.add, tmpf)
            value_v = accs * _recip(wss)
            plsc.store_compressed(out_v.at[pl.ds(i, L)], value_v,
                                  mask=_iota() == 0)
            return 0

        lax.fori_loop(0, rpw, row_body, 0)
        pltpu.sync_copy(out_v.at[pl.ds(0, rpw)], out.at[pl.ds(b_base, rpw)])

    return sc_select


NSPLIT = 2
NB = B // NSPLIT
_sc_select = _make_sc_select(NB)


def kernel(x, W1, b1, Wp, bp, dnd_keys, dnd_vals):
    keys_pad = jnp.pad(dnd_keys, ((0, N_PAD - N), (0, 0)),
                       constant_values=PAD_KEY)
    vals1d = jnp.pad(dnd_vals[:, 0], (0, N_PAD - N))
    hs, ls, vs = [], [], []
    for p in range(NSPLIT):
        h, logits, d2, g3 = _encoder_d2(
            x[p * NB:(p + 1) * NB], W1, b1.reshape(1, H), Wp,
            bp.reshape(1, A), keys_pad, NB)
        d2r = d2.reshape(NB * N_GROUPS, GROUP)
        value = _sc_select(g3, d2r, vals1d)
        hs.append(h)
        ls.append(logits)
        vs.append(value)
    h = jnp.concatenate(hs, axis=0)
    logits = jnp.concatenate(ls, axis=0)
    value = jnp.concatenate(vs, axis=0)
    return (logits, value.reshape(B, 1), h)


# speedup vs baseline: 1.0902x; 1.0557x over previous
import functools

import jax
import jax.numpy as jnp
from jax import lax
from jax.experimental import pallas as pl
from jax.experimental.pallas import tpu as pltpu
from jax.experimental.pallas import tpu_sc as plsc

B, S, H, A, N = 1024, 128, 64, 18, 100000
P_NEIGHBORS = 50
DELTA = 1e-3

GROUP = 128
N_PAD = 100352
N_GROUPS = N_PAD // GROUP
CHUNK = 2048
G_PER_CHUNK = CHUNK // GROUP
N_STEPS = N_PAD // CHUNK
PAD_KEY = 1.0e6


def _encoder_d2_kernel(x_ref, w1_ref, b1_ref, wp_ref, bp_ref, keys_ref,
                       h_ref, logits_ref, d2_ref, g_ref):
    j = pl.program_id(0)

    @pl.when(j == 0)
    def _():
        h0 = jnp.maximum(
            jnp.dot(x_ref[...], w1_ref[...].T, preferred_element_type=jnp.float32)
            + b1_ref[...], 0.0)
        h_ref[...] = h0
        logits_ref[...] = (
            jnp.dot(h0, wp_ref[...].T, preferred_element_type=jnp.float32)
            + bp_ref[...])

    h = h_ref[...]
    hh = jnp.sum(h * h, axis=1, keepdims=True)
    keys = keys_ref[...]
    kk = jnp.sum(keys * keys, axis=1)[None, :]
    s = jnp.dot(h, keys.T, preferred_element_type=jnp.float32)
    d2 = hh - 2.0 * s + kk
    d2_ref[...] = d2
    nb = d2.shape[0]
    gmin = jnp.min(d2.reshape(nb, G_PER_CHUNK, GROUP), axis=-1)
    g_ref[...] = gmin[None]


def _encoder_d2(x, W1, b1, Wp, bp, keys_pad, nb):
    return pl.pallas_call(
        _encoder_d2_kernel,
        grid=(N_STEPS,),
        in_specs=[
            pl.BlockSpec((nb, S), lambda j: (0, 0)),
            pl.BlockSpec((H, S), lambda j: (0, 0)),
            pl.BlockSpec((1, H), lambda j: (0, 0)),
            pl.BlockSpec((A, H), lambda j: (0, 0)),
            pl.BlockSpec((1, A), lambda j: (0, 0)),
            pl.BlockSpec((CHUNK, H), lambda j: (j, 0)),
        ],
        out_specs=[
            pl.BlockSpec((nb, H), lambda j: (0, 0)),
            pl.BlockSpec((nb, A), lambda j: (0, 0)),
            pl.BlockSpec((nb, CHUNK), lambda j: (0, j)),
            pl.BlockSpec((1, nb, G_PER_CHUNK), lambda j: (j, 0, 0)),
        ],
        out_shape=[
            jax.ShapeDtypeStruct((nb, H), jnp.float32),
            jax.ShapeDtypeStruct((nb, A), jnp.float32),
            jax.ShapeDtypeStruct((nb, N_PAD), jnp.float32),
            jax.ShapeDtypeStruct((N_STEPS, nb, G_PER_CHUNK), jnp.float32),
        ],
    )(x, W1, b1, Wp, bp, keys_pad)


NC, NS, L = 2, 16, 16
NW = NC * NS
RPW = B // NW
P = P_NEIGHBORS
ECAP = 1024
GCHUNK = 64
EPHYS = ECAP + GROUP + L
NV_G = N_GROUPS // L
N_GROUPS_REAL = (N + GROUP - 1) // GROUP
INF = float("inf")
PIVOT_ITERS = 7


def _iota():
    return lax.iota(jnp.int32, L)


def _pcount(mask):
    return plsc.all_reduce_population_count(mask)[0]


def _red_vec(x, op, tmp):
    for r in (8, 4, 2, 1):
        tmp[...] = x
        x = op(x, plsc.load_gather(tmp, [_iota() ^ r]))
    return x


def _red(x, op, tmp):
    return _red_vec(x, op, tmp)[0]


def _recip(x):
    u = plsc.bitcast(x, jnp.uint32)
    r = plsc.bitcast(jnp.uint32(0x7EF311C3) - u, jnp.float32)
    for _ in range(3):
        r = r * (2.0 - x * r)
    return r


def _prefix_sum_incl(v, tmp):
    cum = v
    for r in (1, 2, 4, 8):
        tmp[...] = cum
        sh = plsc.load_gather(tmp, [jnp.maximum(_iota() - r, 0)])
        cum = cum + jnp.where(_iota() >= r, sh, 0)
    return cum


def _copy_buf(src, dst, m):
    def body(v, _):
        dst[pl.ds(v * L, L)] = src[pl.ds(v * L, L)]
        return 0
    lax.fori_loop(0, (m + L - 1) // L, body, 0)


def _kth_smallest(wa, wb, m0, k0, tmpf, seed=None):
    def pivot_round(ri, carry):
        m, k, t, done = carry

        def p1(v, acc):
            sv, mnv, mxv = acc
            x = wa[pl.ds(v * L, L)]
            valid = (v * L + _iota()) < m
            sv = sv + jnp.where(valid, x, 0.0)
            mnv = jnp.minimum(mnv, jnp.where(valid, x, INF))
            mxv = jnp.maximum(mxv, jnp.where(valid, x, -INF))
            return (sv, mnv, mxv)
        sv, mnv, mxv = lax.fori_loop(
            0, (m + L - 1) // L, p1,
            (jnp.zeros((L,), jnp.float32), jnp.full((L,), INF, jnp.float32),
             jnp.full((L,), -INF, jnp.float32)))
        msplat = jnp.full((L,), m.astype(jnp.float32))
        mean = (_red_vec(sv, jnp.add, tmpf) * _recip(msplat))[0]
        if seed is not None:
            mean = jnp.where((ri == 0) & seed[1], seed[0], mean)
        mn = _red(mnv, jnp.minimum, tmpf)
        mx = _red(mxv, jnp.maximum, tmpf)

        def p2(v, acc):
            clt, ceq = acc
            x = wa[pl.ds(v * L, L)]
            valid = (v * L + _iota()) < m
            clt = clt + _pcount(valid & (x < mean))
            ceq = ceq + _pcount(valid & (x == mn))
            return (clt, ceq)
        cnt_lt, cnt_eq = lax.fori_loop(
            0, (m + L - 1) // L, p2, (jnp.int32(0), jnp.int32(0)))

        fallback = (cnt_lt == 0) | (cnt_lt == m)
        now_done = (m == k) | (fallback & (k <= cnt_eq))
        t = jnp.where(done, t, jnp.where(m == k, mx, mn))
        done2 = done | now_done

        use_gt = fallback
        use_lt = jnp.logical_not(fallback) & (cnt_lt >= k)

        def p3(v, cnt):
            x = wa[pl.ds(v * L, L)]
            valid = (v * L + _iota()) < m
            keep = jnp.where(use_gt, x > mn,
                             jnp.where(use_lt, x < mean, x >= mean))
            mask = valid & keep & jnp.logical_not(done2)
            plsc.store_compressed(wb.at[pl.ds(cnt, L)], x, mask=mask)
            return cnt + _pcount(mask)
        nm = lax.fori_loop(0, (m + L - 1) // L, p3, jnp.int32(0))
        _copy_buf(wb, wa, nm)

        k2 = jnp.where(done2 | use_lt, k,
                       jnp.where(use_gt, k - cnt_eq, k - cnt_lt))
        m2 = jnp.where(done2, m, nm)
        return (m2, k2, t, done2)

    m0 = jnp.int32(m0)
    k0 = jnp.int32(k0)
    m, k, t, done = lax.fori_loop(
        0, PIVOT_ITERS, pivot_round,
        (m0, k0, jnp.float32(0.0), jnp.bool_(False)))

    sign = jnp.uint32(0x80000000)

    def mapb(v, _):
        x = wa[pl.ds(v * L, L)]
        u = plsc.bitcast(x, jnp.uint32)
        mk = jnp.where(u >= sign, ~u, u | sign)
        wb[pl.ds(v * L, L)] = plsc.bitcast(mk, jnp.float32)
        return 0
    lax.fori_loop(0, (m + L - 1) // L, mapb, 0)

    def bit_round(i, kk_acc):
        cand = kk_acc | (jnp.uint32(1) << (jnp.uint32(31) - i.astype(jnp.uint32)))

        def pc(v, acc):
            mk = plsc.bitcast(wb[pl.ds(v * L, L)], jnp.uint32)
            valid = (v * L + _iota()) < m
            return acc + _pcount(valid & (mk < cand))
        cnt = lax.fori_loop(0, (m + L - 1) // L, pc, jnp.int32(0))
        return jnp.where(cnt < k, cand, kk_acc)

    kbits = lax.fori_loop(0, 32, bit_round, jnp.uint32(0))
    kv = jnp.full((L,), kbits)
    uv = jnp.where(kv >= sign, kv & jnp.uint32(0x7FFFFFFF), ~kv)
    t_bin = plsc.bitcast(uv, jnp.float32)[0]
    return jnp.where(done, t, t_bin)


def _select_exact(ev, ei, en, t2, need, fv, fi, tmpi):
    def pc(v, acc):
        x = ev[pl.ds(v * L, L)]
        valid = (v * L + _iota()) < en
        return acc + _pcount(valid & (x < t2))
    cnt_lt = lax.fori_loop(0, (en + L - 1) // L, pc, jnp.int32(0))
    need_eq = need - cnt_lt

    def body(v, carry):
        fc, eqs = carry
        x = ev[pl.ds(v * L, L)]
        ix = ei[pl.ds(v * L, L)]
        valid = (v * L + _iota()) < en
        m_lt = valid & (x < t2)
        m_eq = valid & (x == t2)
        eq_rank = _prefix_sum_incl(m_eq.astype(jnp.int32), tmpi)
        take_eq = m_eq & ((eqs + eq_rank) <= need_eq)
        mask = m_lt | take_eq
        plsc.store_compressed(fv.at[pl.ds(fc, L)], x, mask=mask)
        plsc.store_compressed(fi.at[pl.ds(fc, L)], ix, mask=mask)
        fc = fc + _pcount(mask)
        eqs = eqs + _pcount(m_eq)
        return (fc, eqs)

    lax.fori_loop(0, (en + L - 1) // L, body, (jnp.int32(0), jnp.int32(0)))


def _make_sc_select(nb):
    mesh = plsc.VectorSubcoreMesh(core_axis_name="c", subcore_axis_name="s",
                                  num_cores=NC, num_subcores=NS)

    @functools.partial(
        pl.kernel,
        out_type=jax.ShapeDtypeStruct((nb,), jnp.float32),
        mesh=mesh,
        compiler_params=pltpu.CompilerParams(needs_layout_passes=False),
        scratch_types=[
            pltpu.VMEM((N_STEPS, 1, G_PER_CHUNK), jnp.float32),
            pltpu.VMEM((N_GROUPS + L,), jnp.float32),
            pltpu.VMEM((EPHYS,), jnp.float32),
            pltpu.VMEM((EPHYS,), jnp.float32),
            pltpu.VMEM((N_GROUPS + L,), jnp.int32),
            pltpu.VMEM((N_GROUPS + L,), jnp.int32),
            pltpu.VMEM((GCHUNK,), jnp.int32),
            pltpu.VMEM((GCHUNK, GROUP), jnp.float32),
            pltpu.VMEM((EPHYS,), jnp.float32),
            pltpu.VMEM((EPHYS,), jnp.int32),
            pltpu.VMEM((5 * L,), jnp.float32),
            pltpu.VMEM((5 * L,), jnp.int32),
            pltpu.VMEM((N_PAD,), jnp.float32),
            pltpu.VMEM((3 * L,), jnp.float32),
            pltpu.VMEM((GCHUNK + L,), jnp.int32),
            pltpu.VMEM((L,), jnp.float32),
            pltpu.VMEM((L,), jnp.int32),
            pltpu.SemaphoreType.DMA,
        ],
    )
    def sc_select(g3, d2r, vals1d, out, grow, wa, wb, wa2, sel_rid, sel_gid,
                  idx_rows, cand_d2, elem_v, elem_i, fv, fi, valsv, out_v,
                  gidbuf, tmpf, tmpi, sem):
        rpw = nb // NW
        wid = lax.axis_index("s") * NC + lax.axis_index("c")
        b_base = wid * rpw
        pltpu.sync_copy(vals1d, valsv)

        def row_body(i, seeds):
            t_prev, t2_prev, has_prev = seeds
            b = b_base + i
            pltpu.sync_copy(g3.at[:, pl.ds(b, 1), :], grow)

            def init_wa(j, _):
                wa[pl.ds(j * L, L)] = grow[j, 0]
                return 0
            lax.fori_loop(0, NV_G, init_wa, 0)

            t = _kth_smallest(wa, wb, N_GROUPS_REAL, P, tmpf,
                              seed=(t_prev, has_prev))

            def sel_body(j, ns):
                x = grow[j, 0]
                rid = b * N_GROUPS + j * L + _iota()
                gid = j * L + _iota()
                mask = x <= t
                plsc.store_compressed(sel_rid.at[pl.ds(ns, L)], rid, mask=mask)
                plsc.store_compressed(sel_gid.at[pl.ds(ns, L)], gid, mask=mask)
                return ns + _pcount(mask)
            ns = lax.fori_loop(0, NV_G, sel_body, jnp.int32(0))

            def chunk_body(c, en):
                for q in range(GCHUNK // L):
                    lanebase = c * GCHUNK + q * L
                    rid = sel_rid[pl.ds(lanebase, L)]
                    gid = sel_gid[pl.ds(lanebase, L)]
                    valid = (lanebase + _iota()) < ns
                    fill = b * N_GROUPS + (q * L) + _iota()
                    idx_rows[pl.ds(q * L, L)] = jnp.where(valid, rid, fill)
                    gidbuf[pl.ds(q * L, L)] = gid
                pltpu.async_copy(d2r.at[idx_rows], cand_d2, sem).wait()

                n_take = jnp.minimum(ns - c * GCHUNK, GCHUNK)

                def group_body(gi, en2):
                    gid = gidbuf[pl.ds(gi, L)][0]
                    ebase = gid * GROUP
                    for sub in range(GROUP // L):
                        x = cand_d2[gi, pl.ds(sub * L, L)]
                        eidx = ebase + sub * L + _iota()
                        mask = x <= t
                        plsc.store_compressed(elem_v.at[pl.ds(en2, L)], x,
                                              mask=mask)
                        plsc.store_compressed(elem_i.at[pl.ds(en2, L)], eidx,
                                              mask=mask)
                        en2 = en2 + _pcount(mask)
                    return jnp.minimum(en2, jnp.int32(ECAP))

                return lax.fori_loop(0, n_take, group_body, en)

            en = lax.fori_loop(0, (ns + GCHUNK - 1) // GCHUNK,
                               chunk_body, jnp.int32(0))

            for q in range(5):
                fv[pl.ds(q * L, L)] = jnp.full((L,), INF, jnp.float32)
                fi[pl.ds(q * L, L)] = jnp.zeros((L,), jnp.int32)
            _copy_buf(elem_v, wa2, en)
            t2 = _kth_smallest(wa2, wb, en, P, tmpf,
                               seed=(t2_prev, has_prev))
            _select_exact(elem_v, elem_i, en, t2, P, fv, fi, tmpi)

            acc = jnp.zeros((L,), jnp.float32)
            wsum = jnp.zeros((L,), jnp.float32)
            for q in range(4):
                d = fv[pl.ds(q * L, L)]
                lane_ok = (q * L + _iota()) < P
                w = jnp.where(lane_ok, _recip(d + DELTA), 0.0)
                v = plsc.load_gather(valsv, [fi[pl.ds(q * L, L)]])
                acc = acc + w * v
                wsum = wsum + w
            accs = _red_vec(acc, jnp.add, tmpf)
            wss = _red_vec(wsum, jnp.add, tmpf)
            value_v = accs * _recip(wss)
            plsc.store_compressed(out_v.at[pl.ds(i, L)], value_v,
                                  mask=_iota() == 0)
            return (t, t2, jnp.bool_(True))

        lax.fori_loop(0, rpw, row_body,
                      (jnp.float32(0.0), jnp.float32(0.0), jnp.bool_(False)))
        pltpu.sync_copy(out_v.at[pl.ds(0, rpw)], out.at[pl.ds(b_base, rpw)])

    return sc_select


NSPLIT = 2
NB = B // NSPLIT
_sc_select = _make_sc_select(NB)


def kernel(x, W1, b1, Wp, bp, dnd_keys, dnd_vals):
    keys_pad = jnp.pad(dnd_keys, ((0, N_PAD - N), (0, 0)),
                       constant_values=PAD_KEY)
    vals1d = jnp.pad(dnd_vals[:, 0], (0, N_PAD - N))
    hs, ls, vs = [], [], []
    for p in range(NSPLIT):
        h, logits, d2, g3 = _encoder_d2(
            x[p * NB:(p + 1) * NB], W1, b1.reshape(1, H), Wp,
            bp.reshape(1, A), keys_pad, NB)
        d2r = d2.reshape(NB * N_GROUPS, GROUP)
        value = _sc_select(g3, d2r, vals1d)
        hs.append(h)
        ls.append(logits)
        vs.append(value)
    h = jnp.concatenate(hs, axis=0)
    logits = jnp.concatenate(ls, axis=0)
    value = jnp.concatenate(vs, axis=0)
    return (logits, value.reshape(B, 1), h)
